# Initial kernel scaffold; baseline (speedup 1.0000x reference)
import jax
import jax.numpy as jnp
from jax.experimental import pallas as pl


def kernel(h, x, edge_index, edge_attr, coord_diff, flags, edge_mask, W1, b1, W2, b2, W3):
    raise NotImplementedError("write your pallas kernel here")


# speedup vs baseline: 3.7726x; 3.7726x over previous
import functools

import jax
import jax.numpy as jnp
from jax import lax
from jax.experimental import pallas as pl
from jax.experimental.pallas import tpu as pltpu
from jax.experimental.pallas import tpu_sc as plsc

_COORDS_RANGE = 15.0
_NC = 2
_NS = 16
_NW = _NC * _NS
_GCHUNK = 128


def _silu(z):
    return z * jax.nn.sigmoid(z)


def _proj_body(h_ref, wab_ref, a_ref, b_ref):
    hb = h_ref[...]
    a_ref[...] = jnp.dot(hb, wab_ref[0], preferred_element_type=jnp.float32)
    b_ref[...] = jnp.dot(hb, wab_ref[1], preferred_element_type=jnp.float32)


def _node_proj(h, w_ab, n_blk):
    n, hdim = h.shape
    grid = n // n_blk
    return pl.pallas_call(
        _proj_body,
        grid=(grid,),
        in_specs=[
            pl.BlockSpec((n_blk, hdim), lambda i: (i, 0)),
            pl.BlockSpec((2, hdim, hdim), lambda i: (0, 0, 0)),
        ],
        out_specs=[
            pl.BlockSpec((n_blk, hdim), lambda i: (i, 0)),
            pl.BlockSpec((n_blk, hdim), lambda i: (i, 0)),
        ],
        out_shape=[
            jax.ShapeDtypeStruct((n, hdim), jnp.float32),
            jax.ShapeDtypeStruct((n, hdim), jnp.float32),
        ],
    )(h, w_ab)


def _gather_sum(a_tab, b_tab, row, col):
    e = row.shape[0]
    hdim = a_tab.shape[1]
    nchunks_total = e // _GCHUNK
    mesh = plsc.VectorSubcoreMesh(core_axis_name="c", subcore_axis_name="s")

    @functools.partial(
        pl.kernel,
        mesh=mesh,
        compiler_params=pltpu.CompilerParams(needs_layout_passes=False),
        out_type=jax.ShapeDtypeStruct((e, hdim), jnp.float32),
        scratch_types=[
            pltpu.VMEM((_GCHUNK,), jnp.int32),
            pltpu.VMEM((_GCHUNK,), jnp.int32),
            pltpu.VMEM((_GCHUNK, hdim), jnp.float32),
            pltpu.VMEM((_GCHUNK, hdim), jnp.float32),
            pltpu.SemaphoreType.DMA,
            pltpu.SemaphoreType.DMA,
        ],
    )
    def k(a_hbm, b_hbm, row_hbm, col_hbm, s_hbm, idxr, idxc, bufa, bufb,
          sema, semb):
        wid = lax.axis_index("s") * _NC + lax.axis_index("c")
        my_chunks = (nchunks_total - wid + _NW - 1) // _NW

        def chunk_body(i, _):
            eb = (wid + i * _NW) * _GCHUNK
            pltpu.sync_copy(row_hbm.at[pl.ds(eb, _GCHUNK)], idxr)
            pltpu.sync_copy(col_hbm.at[pl.ds(eb, _GCHUNK)], idxc)
            cpa = pltpu.async_copy(a_hbm.at[idxr], bufa, sema)
            cpb = pltpu.async_copy(b_hbm.at[idxc], bufb, semb)
            cpa.wait()
            cpb.wait()

            def add_body(r, _):
                for kk in range(hdim // 16):
                    sl = pl.ds(kk * 16, 16)
                    bufa[r, sl] = bufa[r, sl] + bufb[r, sl]
                return 0

            lax.fori_loop(0, _GCHUNK, add_body, 0)
            pltpu.sync_copy(bufa, s_hbm.at[pl.ds(eb, _GCHUNK)])
            return 0

        lax.fori_loop(0, my_chunks, chunk_body, 0)

    return k(a_tab, b_tab, row, col)


def _mlp_body(s_ref, ea_ref, w1c_ref, b1_ref, w2_ref, b2_ref, w3_ref, g_ref):
    z = s_ref[...] + jnp.dot(ea_ref[...], w1c_ref[...],
                             preferred_element_type=jnp.float32) + b1_ref[...]
    t = _silu(z)
    z2 = jnp.dot(t, w2_ref[...], preferred_element_type=jnp.float32) + b2_ref[...]
    t2 = _silu(z2)
    s = jnp.dot(t2, w3_ref[...], preferred_element_type=jnp.float32)
    g_ref[...] = jnp.tanh(s) * _COORDS_RANGE


def _edge_mlp(s_sum, edge_attr, w1c, b1, w2, b2, w3, e_blk):
    e, hdim = s_sum.shape
    ed = edge_attr.shape[1]
    grid = e // e_blk
    return pl.pallas_call(
        _mlp_body,
        grid=(grid,),
        in_specs=[
            pl.BlockSpec((e_blk, hdim), lambda i: (i, 0)),
            pl.BlockSpec((e_blk, ed), lambda i: (i, 0)),
            pl.BlockSpec((ed, hdim), lambda i: (0, 0)),
            pl.BlockSpec((1, hdim), lambda i: (0, 0)),
            pl.BlockSpec((hdim, hdim), lambda i: (0, 0)),
            pl.BlockSpec((1, hdim), lambda i: (0, 0)),
            pl.BlockSpec((hdim, 1), lambda i: (0, 0)),
        ],
        out_specs=pl.BlockSpec((e_blk, 1), lambda i: (i, 0)),
        out_shape=jax.ShapeDtypeStruct((e, 1), jnp.float32),
    )(s_sum, edge_attr, w1c, b1, w2, b2, w3)


def _scatter_partials(row, g_flat, cd_flat, n):
    e = row.shape[0]
    epw = e // _NW
    mesh = plsc.VectorSubcoreMesh(core_axis_name="c", subcore_axis_name="s")

    @functools.partial(
        pl.kernel,
        mesh=mesh,
        compiler_params=pltpu.CompilerParams(needs_layout_passes=False),
        out_type=jax.ShapeDtypeStruct((_NW, n * 3), jnp.float32),
        scratch_types=[
            pltpu.VMEM((epw,), jnp.int32),
            pltpu.VMEM((epw,), jnp.float32),
            pltpu.VMEM((epw * 3,), jnp.float32),
            pltpu.VMEM((n * 3,), jnp.float32),
        ],
    )
    def k(row_hbm, g_hbm, cd_hbm, part_hbm, row_v, g_v, cd_v, acc):
        wid = lax.axis_index("s") * _NC + lax.axis_index("c")
        base = wid * epw
        pltpu.sync_copy(row_hbm.at[pl.ds(base, epw)], row_v)
        pltpu.sync_copy(g_hbm.at[pl.ds(base, epw)], g_v)
        pltpu.sync_copy(cd_hbm.at[pl.ds(base * 3, epw * 3)], cd_v)

        zeros16 = jnp.zeros((16,), jnp.float32)

        def z_body(i, _):
            acc[pl.ds(i * 16, 16)] = zeros16
            return 0

        lax.fori_loop(0, (n * 3) // 16, z_body, 0)

        lane = jnp.arange(16, dtype=jnp.int32)

        def e_body(i, _):
            off = i * 16
            idx = row_v[pl.ds(off, 16)]
            gv = g_v[pl.ds(off, 16)]
            tgt = idx * 3
            src = off * 3 + lane * 3
            for c in range(3):
                cdv = plsc.load_gather(cd_v, [src + c])
                plsc.addupdate_scatter(acc, [tgt + c], cdv * gv)
            return 0

        lax.fori_loop(0, epw // 16, e_body, 0)
        pltpu.sync_copy(acc, part_hbm.at[wid])

    return k(row, g_flat, cd_flat)


def _reduce_body(p_ref, x_ref, f_ref, o_ref):
    o_ref[...] = (x_ref[...] + jnp.sum(p_ref[...], axis=0, keepdims=True)) \
        * f_ref[...]


def _final_reduce(partials, x_flat, flags_flat):
    n3 = x_flat.shape[1]
    return pl.pallas_call(
        _reduce_body,
        in_specs=[
            pl.BlockSpec((_NW, n3), lambda: (0, 0)),
            pl.BlockSpec((1, n3), lambda: (0, 0)),
            pl.BlockSpec((1, n3), lambda: (0, 0)),
        ],
        out_specs=pl.BlockSpec((1, n3), lambda: (0, 0)),
        out_shape=jax.ShapeDtypeStruct((1, n3), jnp.float32),
    )(partials, x_flat, flags_flat)


def kernel(h, x, edge_index, edge_attr, coord_diff, flags, edge_mask,
           W1, b1, W2, b2, W3):
    n, hdim = h.shape
    e = edge_index.shape[1]
    row = edge_index[0]
    col = edge_index[1]

    w_ab = jnp.stack([W1[:hdim], W1[hdim:2 * hdim]])
    w1c = W1[2 * hdim:]

    a_tab, b_tab = _node_proj(h, w_ab, 1000)
    s_sum = _gather_sum(a_tab, b_tab, row, col)
    g = _edge_mlp(s_sum, edge_attr, w1c, b1.reshape(1, -1),
                  W2, b2.reshape(1, -1), W3, 1600)
    partials = _scatter_partials(row, g.reshape(-1),
                                 coord_diff.reshape(-1), n)
    out = _final_reduce(partials, x.reshape(1, -1),
                        jnp.broadcast_to(flags, (n, 3)).reshape(1, -1))
    return out.reshape(n, 3)


# speedup vs baseline: 3.8436x; 1.0188x over previous
import functools

import jax
import jax.numpy as jnp
from jax import lax
from jax.experimental import pallas as pl
from jax.experimental.pallas import tpu as pltpu
from jax.experimental.pallas import tpu_sc as plsc

_COORDS_RANGE = 15.0
_NC = 2
_NS = 16
_NW = _NC * _NS
_GCHUNK = 128


def _silu(z):
    return z * jax.nn.sigmoid(z)


def _proj_body(h_ref, wab_ref, a_ref, b_ref):
    hb = h_ref[...]
    a_ref[...] = jnp.dot(hb, wab_ref[0], preferred_element_type=jnp.float32)
    b_ref[...] = jnp.dot(hb, wab_ref[1], preferred_element_type=jnp.float32)


def _node_proj(h, w_ab, n_blk):
    n, hdim = h.shape
    grid = n // n_blk
    return pl.pallas_call(
        _proj_body,
        grid=(grid,),
        in_specs=[
            pl.BlockSpec((n_blk, hdim), lambda i: (i, 0)),
            pl.BlockSpec((2, hdim, hdim), lambda i: (0, 0, 0)),
        ],
        out_specs=[
            pl.BlockSpec((n_blk, hdim), lambda i: (i, 0)),
            pl.BlockSpec((n_blk, hdim), lambda i: (i, 0)),
        ],
        out_shape=[
            jax.ShapeDtypeStruct((n, hdim), jnp.float32),
            jax.ShapeDtypeStruct((n, hdim), jnp.float32),
        ],
    )(h, w_ab)


def _gather_sum(a_tab, b_tab, row, col):
    e = row.shape[0]
    hdim = a_tab.shape[1]
    nchunks_total = e // _GCHUNK
    npairs = (nchunks_total + 2 * _NW - 1) // (2 * _NW)
    mesh = plsc.VectorSubcoreMesh(core_axis_name="c", subcore_axis_name="s")

    @functools.partial(
        pl.kernel,
        mesh=mesh,
        compiler_params=pltpu.CompilerParams(needs_layout_passes=False),
        out_type=jax.ShapeDtypeStruct((e, hdim), jnp.float32),
        scratch_types=[
            pltpu.VMEM((_GCHUNK,), jnp.int32),
            pltpu.VMEM((_GCHUNK,), jnp.int32),
            pltpu.VMEM((_GCHUNK,), jnp.int32),
            pltpu.VMEM((_GCHUNK,), jnp.int32),
            pltpu.VMEM((_GCHUNK, hdim), jnp.float32),
            pltpu.VMEM((_GCHUNK, hdim), jnp.float32),
            pltpu.VMEM((_GCHUNK, hdim), jnp.float32),
            pltpu.VMEM((_GCHUNK, hdim), jnp.float32),
            pltpu.SemaphoreType.DMA,
            pltpu.SemaphoreType.DMA,
            pltpu.SemaphoreType.DMA,
            pltpu.SemaphoreType.DMA,
        ],
    )
    def k(a_hbm, b_hbm, row_hbm, col_hbm, s_hbm,
          idxr0, idxc0, idxr1, idxc1, bufa0, bufb0, bufa1, bufb1,
          sem0, sem1, semw0, semw1):
        wid = lax.axis_index("s") * _NC + lax.axis_index("c")

        bufs = ((idxr0, idxc0, bufa0, bufb0, sem0, semw0),
                (idxr1, idxc1, bufa1, bufb1, sem1, semw1))

        def issue(cid, parity, drain_write):
            idxr, idxc, bufa, bufb, sem, semw = bufs[parity]

            @pl.when(cid < nchunks_total)
            def _():
                if drain_write:
                    pltpu.make_async_copy(
                        s_hbm.at[pl.ds(0, _GCHUNK)], bufa, semw).wait()
                eb = cid * _GCHUNK
                pltpu.sync_copy(row_hbm.at[pl.ds(eb, _GCHUNK)], idxr)
                pltpu.sync_copy(col_hbm.at[pl.ds(eb, _GCHUNK)], idxc)
                pltpu.async_copy(a_hbm.at[idxr], bufa, sem)
                pltpu.async_copy(b_hbm.at[idxc], bufb, sem)

        def process(cid, parity):
            idxr, idxc, bufa, bufb, sem, semw = bufs[parity]

            @pl.when(cid < nchunks_total)
            def _():
                pltpu.make_async_copy(a_hbm.at[idxr], bufa, sem).wait()
                pltpu.make_async_copy(b_hbm.at[idxc], bufb, sem).wait()

                def add_body(r, _):
                    for kk in range(hdim // 16):
                        sl = pl.ds(kk * 16, 16)
                        bufa[r, sl] = bufa[r, sl] + bufb[r, sl]
                    return 0

                lax.fori_loop(0, _GCHUNK, add_body, 0)
                eb = cid * _GCHUNK
                pltpu.async_copy(bufa, s_hbm.at[pl.ds(eb, _GCHUNK)], semw)

        issue(wid, 0, False)
        issue(wid + _NW, 1, False)
        process(wid, 0)
        issue(wid + 2 * _NW, 0, True)
        process(wid + _NW, 1)

        def pair_body(j, _):
            c1 = wid + (2 * j + 1) * _NW
            c0n = wid + (2 * j + 2) * _NW
            issue(c1, 1, True)
            process(wid + (2 * j) * _NW, 0)
            issue(c0n, 0, True)
            process(c1, 1)
            return 0

        lax.fori_loop(1, npairs, pair_body, 0)
        pltpu.make_async_copy(
            s_hbm.at[pl.ds(0, _GCHUNK)], bufa0, semw0).wait()
        pltpu.make_async_copy(
            s_hbm.at[pl.ds(0, _GCHUNK)], bufa1, semw1).wait()

    return k(a_tab, b_tab, row, col)


def _mlp_body(s_ref, ea_ref, w1c_ref, b1_ref, w2_ref, b2_ref, w3_ref, g_ref):
    z = s_ref[...] + jnp.dot(ea_ref[...], w1c_ref[...],
                             preferred_element_type=jnp.float32) + b1_ref[...]
    t = _silu(z).astype(jnp.bfloat16)
    z2 = jnp.dot(t, w2_ref[...], preferred_element_type=jnp.float32) \
        + b2_ref[...]
    t2 = _silu(z2)
    s = jnp.dot(t2, w3_ref[...], preferred_element_type=jnp.float32)
    g_ref[...] = jnp.tanh(s) * _COORDS_RANGE


def _edge_mlp(s_sum, edge_attr, w1c, b1, w2, b2, w3, e_blk):
    e, hdim = s_sum.shape
    ed = edge_attr.shape[1]
    grid = e // e_blk
    return pl.pallas_call(
        _mlp_body,
        grid=(grid,),
        in_specs=[
            pl.BlockSpec((e_blk, hdim), lambda i: (i, 0)),
            pl.BlockSpec((e_blk, ed), lambda i: (i, 0)),
            pl.BlockSpec((ed, hdim), lambda i: (0, 0)),
            pl.BlockSpec((1, hdim), lambda i: (0, 0)),
            pl.BlockSpec((hdim, hdim), lambda i: (0, 0)),
            pl.BlockSpec((1, hdim), lambda i: (0, 0)),
            pl.BlockSpec((hdim, 1), lambda i: (0, 0)),
        ],
        out_specs=pl.BlockSpec((e_blk, 1), lambda i: (i, 0)),
        out_shape=jax.ShapeDtypeStruct((e, 1), jnp.float32),
    )(s_sum, edge_attr, w1c, b1, w2, b2, w3)


def _scatter_partials(row, g_flat, cd_flat, n):
    e = row.shape[0]
    epw = e // _NW
    mesh = plsc.VectorSubcoreMesh(core_axis_name="c", subcore_axis_name="s")

    @functools.partial(
        pl.kernel,
        mesh=mesh,
        compiler_params=pltpu.CompilerParams(needs_layout_passes=False),
        out_type=jax.ShapeDtypeStruct((_NW, n * 3), jnp.float32),
        scratch_types=[
            pltpu.VMEM((epw,), jnp.int32),
            pltpu.VMEM((epw,), jnp.float32),
            pltpu.VMEM((epw * 3,), jnp.float32),
            pltpu.VMEM((n * 3,), jnp.float32),
        ],
    )
    def k(row_hbm, g_hbm, cd_hbm, part_hbm, row_v, g_v, cd_v, acc):
        wid = lax.axis_index("s") * _NC + lax.axis_index("c")
        base = wid * epw
        pltpu.sync_copy(row_hbm.at[pl.ds(base, epw)], row_v)
        pltpu.sync_copy(g_hbm.at[pl.ds(base, epw)], g_v)
        pltpu.sync_copy(cd_hbm.at[pl.ds(base * 3, epw * 3)], cd_v)

        zeros16 = jnp.zeros((16,), jnp.float32)

        def z_body(i, _):
            acc[pl.ds(i * 16, 16)] = zeros16
            return 0

        lax.fori_loop(0, (n * 3) // 16, z_body, 0)

        lane = jnp.arange(16, dtype=jnp.int32)

        def e_body(i, _):
            off = i * 16
            idx = row_v[pl.ds(off, 16)]
            gv = g_v[pl.ds(off, 16)]
            tgt = idx * 3
            src = off * 3 + lane * 3
            for c in range(3):
                cdv = plsc.load_gather(cd_v, [src + c])
                plsc.addupdate_scatter(acc, [tgt + c], cdv * gv)
            return 0

        lax.fori_loop(0, epw // 16, e_body, 0)
        pltpu.sync_copy(acc, part_hbm.at[wid])

    return k(row, g_flat, cd_flat)


def _reduce_body(p_ref, x_ref, f_ref, o_ref):
    o_ref[...] = (x_ref[...] + jnp.sum(p_ref[...], axis=0, keepdims=True)) \
        * f_ref[...]


def _final_reduce(partials, x_flat, flags_flat):
    n3 = x_flat.shape[1]
    return pl.pallas_call(
        _reduce_body,
        in_specs=[
            pl.BlockSpec((_NW, n3), lambda: (0, 0)),
            pl.BlockSpec((1, n3), lambda: (0, 0)),
            pl.BlockSpec((1, n3), lambda: (0, 0)),
        ],
        out_specs=pl.BlockSpec((1, n3), lambda: (0, 0)),
        out_shape=jax.ShapeDtypeStruct((1, n3), jnp.float32),
    )(partials, x_flat, flags_flat)


def kernel(h, x, edge_index, edge_attr, coord_diff, flags, edge_mask,
           W1, b1, W2, b2, W3):
    n, hdim = h.shape
    e = edge_index.shape[1]
    row = edge_index[0]
    col = edge_index[1]

    w_ab = jnp.stack([W1[:hdim], W1[hdim:2 * hdim]])
    w1c = W1[2 * hdim:]

    a_tab, b_tab = _node_proj(h, w_ab, 1000)
    s_sum = _gather_sum(a_tab, b_tab, row, col)
    g = _edge_mlp(s_sum, edge_attr, w1c, b1.reshape(1, -1),
                  W2.astype(jnp.bfloat16), b2.reshape(1, -1), W3, 1600)
    partials = _scatter_partials(row, g.reshape(-1),
                                 coord_diff.reshape(-1), n)
    out = _final_reduce(partials, x.reshape(1, -1),
                        jnp.broadcast_to(flags, (n, 3)).reshape(1, -1))
    return out.reshape(n, 3)


# speedup vs baseline: 7.2855x; 1.8955x over previous
import functools

import jax
import jax.numpy as jnp
from jax import lax
from jax.experimental import pallas as pl
from jax.experimental.pallas import tpu as pltpu
from jax.experimental.pallas import tpu_sc as plsc

_COORDS_RANGE = 15.0
_NC = 2
_NS = 16
_NW = _NC * _NS
_GCHUNK = 128
_NSEG = 4


def _silu(z):
    return z * jax.nn.sigmoid(z)


def _proj_body(h_ref, wab_ref, a_ref, b_ref):
    hb = h_ref[...]
    a_ref[...] = jnp.dot(hb, wab_ref[0], preferred_element_type=jnp.float32)
    b_ref[...] = jnp.dot(hb, wab_ref[1], preferred_element_type=jnp.float32)


def _node_proj(h, w_ab, n_blk):
    n, hdim = h.shape
    grid = n // n_blk
    return pl.pallas_call(
        _proj_body,
        grid=(grid,),
        in_specs=[
            pl.BlockSpec((n_blk, hdim), lambda i: (i, 0)),
            pl.BlockSpec((2, hdim, hdim), lambda i: (0, 0, 0)),
        ],
        out_specs=[
            pl.BlockSpec((n_blk, hdim), lambda i: (i, 0)),
            pl.BlockSpec((n_blk, hdim), lambda i: (i, 0)),
        ],
        out_shape=[
            jax.ShapeDtypeStruct((n, hdim), jnp.float32),
            jax.ShapeDtypeStruct((n, hdim), jnp.float32),
        ],
    )(h, w_ab)


def _gather_sum(a_tab, b_tab, row, col):
    e = row.shape[0]
    hdim = a_tab.shape[1]
    nchunks_total = e // _GCHUNK
    npairs = (nchunks_total + 2 * _NW - 1) // (2 * _NW)
    mesh = plsc.VectorSubcoreMesh(core_axis_name="c", subcore_axis_name="s")

    @functools.partial(
        pl.kernel,
        mesh=mesh,
        compiler_params=pltpu.CompilerParams(needs_layout_passes=False),
        out_type=jax.ShapeDtypeStruct((e, hdim), jnp.float32),
        scratch_types=[
            pltpu.VMEM((_GCHUNK,), jnp.int32),
            pltpu.VMEM((_GCHUNK,), jnp.int32),
            pltpu.VMEM((_GCHUNK,), jnp.int32),
            pltpu.VMEM((_GCHUNK,), jnp.int32),
            pltpu.VMEM((_GCHUNK, hdim), jnp.float32),
            pltpu.VMEM((_GCHUNK, hdim), jnp.float32),
            pltpu.VMEM((_GCHUNK, hdim), jnp.float32),
            pltpu.VMEM((_GCHUNK, hdim), jnp.float32),
            pltpu.SemaphoreType.DMA,
            pltpu.SemaphoreType.DMA,
            pltpu.SemaphoreType.DMA,
            pltpu.SemaphoreType.DMA,
        ],
    )
    def k(a_hbm, b_hbm, row_hbm, col_hbm, s_hbm,
          idxr0, idxc0, idxr1, idxc1, bufa0, bufb0, bufa1, bufb1,
          sem0, sem1, semw0, semw1):
        wid = lax.axis_index("s") * _NC + lax.axis_index("c")

        bufs = ((idxr0, idxc0, bufa0, bufb0, sem0, semw0),
                (idxr1, idxc1, bufa1, bufb1, sem1, semw1))

        def issue(cid, parity, drain_write):
            idxr, idxc, bufa, bufb, sem, semw = bufs[parity]

            @pl.when(cid < nchunks_total)
            def _():
                if drain_write:
                    pltpu.make_async_copy(
                        s_hbm.at[pl.ds(0, _GCHUNK)], bufa, semw).wait()
                eb = cid * _GCHUNK
                pltpu.sync_copy(row_hbm.at[pl.ds(eb, _GCHUNK)], idxr)
                pltpu.sync_copy(col_hbm.at[pl.ds(eb, _GCHUNK)], idxc)
                pltpu.async_copy(a_hbm.at[idxr], bufa, sem)
                pltpu.async_copy(b_hbm.at[idxc], bufb, sem)

        def process(cid, parity):
            idxr, idxc, bufa, bufb, sem, semw = bufs[parity]

            @pl.when(cid < nchunks_total)
            def _():
                pltpu.make_async_copy(a_hbm.at[idxr], bufa, sem).wait()
                pltpu.make_async_copy(b_hbm.at[idxc], bufb, sem).wait()

                def add_body(r, _):
                    for kk in range(hdim // 16):
                        sl = pl.ds(kk * 16, 16)
                        bufa[r, sl] = bufa[r, sl] + bufb[r, sl]
                    return 0

                lax.fori_loop(0, _GCHUNK, add_body, 0)
                eb = cid * _GCHUNK
                pltpu.async_copy(bufa, s_hbm.at[pl.ds(eb, _GCHUNK)], semw)

        issue(wid, 0, False)
        issue(wid + _NW, 1, False)
        process(wid, 0)
        issue(wid + 2 * _NW, 0, True)
        process(wid + _NW, 1)

        def pair_body(j, _):
            c1 = wid + (2 * j + 1) * _NW
            c0n = wid + (2 * j + 2) * _NW
            issue(c1, 1, True)
            process(wid + (2 * j) * _NW, 0)
            issue(c0n, 0, True)
            process(c1, 1)
            return 0

        lax.fori_loop(1, npairs, pair_body, 0)
        pltpu.make_async_copy(
            s_hbm.at[pl.ds(0, _GCHUNK)], bufa0, semw0).wait()
        pltpu.make_async_copy(
            s_hbm.at[pl.ds(0, _GCHUNK)], bufa1, semw1).wait()

    return k(a_tab, b_tab, row, col)


def _mlp_body(s_ref, eat_ref, w1c_ref, b1_ref, w2_ref, b2_ref, w3_ref, g_ref):
    zc = jax.lax.dot_general(eat_ref[...], w1c_ref[...],
                             (((0,), (0,)), ((), ())),
                             preferred_element_type=jnp.float32)
    z = s_ref[...] + zc + b1_ref[...]
    t = _silu(z).astype(jnp.bfloat16)
    z2 = jnp.dot(t, w2_ref[...], preferred_element_type=jnp.float32) \
        + b2_ref[...]
    t2 = _silu(z2)
    s = jnp.sum(t2 * w3_ref[...], axis=1)
    e_blk = s.shape[0]
    g_ref[pl.ds(pl.program_id(0) * e_blk, e_blk)] = \
        jnp.tanh(s) * _COORDS_RANGE


def _edge_mlp(s_sum, ea_t, seg_off, w1c, b1, w2, b2, w3, e_blk):
    es, hdim = s_sum.shape
    ed = ea_t.shape[0]
    grid = es // e_blk
    blk_off = seg_off // e_blk
    return pl.pallas_call(
        _mlp_body,
        grid=(grid,),
        in_specs=[
            pl.BlockSpec((e_blk, hdim), lambda i: (i, 0)),
            pl.BlockSpec((ed, e_blk), lambda i: (0, i + blk_off)),
            pl.BlockSpec((ed, hdim), lambda i: (0, 0)),
            pl.BlockSpec((1, hdim), lambda i: (0, 0)),
            pl.BlockSpec((hdim, hdim), lambda i: (0, 0)),
            pl.BlockSpec((1, hdim), lambda i: (0, 0)),
            pl.BlockSpec((1, hdim), lambda i: (0, 0)),
        ],
        out_specs=pl.BlockSpec((es,), lambda i: (0,)),
        out_shape=jax.ShapeDtypeStruct((es,), jnp.float32),
    )(s_sum, ea_t, w1c, b1, w2, b2, w3)


def _scatter_partials(row, g_flat, cd_t, n):
    e = row.shape[0]
    epw = e // _NW
    cdw = (epw // 128 + 1) * 128
    mesh = plsc.VectorSubcoreMesh(core_axis_name="c", subcore_axis_name="s")

    @functools.partial(
        pl.kernel,
        mesh=mesh,
        compiler_params=pltpu.CompilerParams(needs_layout_passes=False),
        out_type=jax.ShapeDtypeStruct((_NW, n * 3), jnp.float32),
        scratch_types=[
            pltpu.VMEM((epw,), jnp.int32),
            pltpu.VMEM((epw,), jnp.float32),
            pltpu.VMEM((3, cdw), jnp.float32),
            pltpu.VMEM((n * 3,), jnp.float32),
        ],
    )
    def k(row_hbm, g_hbm, cd_hbm, part_hbm, row_v, g_v, cdv, acc):
        wid = lax.axis_index("s") * _NC + lax.axis_index("c")
        base = wid * epw
        rem = lax.rem(base, 128)
        base_al = pl.multiple_of(base - rem, 128)
        pltpu.sync_copy(row_hbm.at[pl.ds(base, epw)], row_v)
        pltpu.sync_copy(g_hbm.at[pl.ds(base, epw)], g_v)
        pltpu.sync_copy(cd_hbm.at[pl.ds(0, 3), pl.ds(base_al, cdw)], cdv)

        zeros16 = jnp.zeros((16,), jnp.float32)

        def z_body(i, _):
            acc[pl.ds(i * 16, 16)] = zeros16
            return 0

        lax.fori_loop(0, (n * 3) // 16, z_body, 0)

        def e_body(i, _):
            off = i * 16
            sl = pl.ds(off, 16)
            idx = row_v[sl]
            gv = g_v[sl]
            tgt = idx * 3
            for c in range(3):
                plsc.addupdate_scatter(
                    acc, [tgt + c], cdv[c, pl.ds(rem + off, 16)] * gv)
            return 0

        lax.fori_loop(0, epw // 16, e_body, 0)
        pltpu.sync_copy(acc, part_hbm.at[wid])

    return k(row, g_flat, cd_t)


def _reduce_body(p_ref, x_ref, f_ref, o_ref):
    o_ref[...] = (x_ref[...] + jnp.sum(p_ref[...], axis=0, keepdims=True)) \
        * f_ref[...]


def _final_reduce(partials, x_flat, flags_flat):
    n3 = x_flat.shape[1]
    return pl.pallas_call(
        _reduce_body,
        in_specs=[
            pl.BlockSpec((_NW, n3), lambda: (0, 0)),
            pl.BlockSpec((1, n3), lambda: (0, 0)),
            pl.BlockSpec((1, n3), lambda: (0, 0)),
        ],
        out_specs=pl.BlockSpec((1, n3), lambda: (0, 0)),
        out_shape=jax.ShapeDtypeStruct((1, n3), jnp.float32),
    )(partials, x_flat, flags_flat)


def kernel(h, x, edge_index, edge_attr, coord_diff, flags, edge_mask,
           W1, b1, W2, b2, W3):
    n, hdim = h.shape
    e = edge_index.shape[1]
    row = edge_index[0]
    col = edge_index[1]

    w_ab = jnp.stack([W1[:hdim], W1[hdim:2 * hdim]])
    w1c = W1[2 * hdim:]
    ea_t = edge_attr.T
    cd_t = coord_diff.T

    a_tab, b_tab = _node_proj(h, w_ab, 1000)

    es = e // _NSEG
    gs = []
    for sg in range(_NSEG):
        s_sum = _gather_sum(a_tab, b_tab,
                            lax.slice(row, (sg * es,), ((sg + 1) * es,)),
                            lax.slice(col, (sg * es,), ((sg + 1) * es,)))
        gs.append(_edge_mlp(s_sum, ea_t, sg * es, w1c, b1.reshape(1, -1),
                            W2.astype(jnp.bfloat16), b2.reshape(1, -1),
                            W3.reshape(1, -1), 3200))
    g = jnp.concatenate(gs, axis=0)

    partials = _scatter_partials(row, g, cd_t, n)
    out = _final_reduce(partials, x.reshape(1, -1),
                        jnp.broadcast_to(flags, (n, 3)).reshape(1, -1))
    return out.reshape(n, 3)
